# baseline (speedup 1.0000x reference)
import functools

import jax
import jax.numpy as jnp
from jax.experimental import pallas as pl
from jax.experimental.pallas import tpu as pltpu

B, S, H = 1, 2048, 768
E, K, DFF = 8, 2, 1024
T = B * S
ROUTER_AUX_COEF = 0.001
ROUTER_Z_COEF = 0.001


def _router_body(x_ref, w_ref, b_ref, tp_ref, ti_ref, comb_ref, aux_ref):
    xb = x_ref[...].astype(jnp.bfloat16)
    wb = w_ref[...].astype(jnp.bfloat16)
    logits = jax.lax.dot_general(
        xb, wb, (((1,), (0,)), ((), ())),
        preferred_element_type=jnp.float32) + b_ref[...][None, :]
    m = jnp.max(logits, axis=-1, keepdims=True)
    ex = jnp.exp(logits - m)
    z = jnp.sum(ex, axis=-1, keepdims=True)
    p = ex / z

    lane = jax.lax.broadcasted_iota(jnp.int32, (T, E), 1)
    v1 = jnp.max(p, axis=-1, keepdims=True)
    i1 = jnp.min(jnp.where(p == v1, lane, E), axis=-1, keepdims=True)
    p_m = jnp.where(lane == i1, -jnp.inf, p)
    v2 = jnp.max(p_m, axis=-1, keepdims=True)
    i2 = jnp.min(jnp.where(p_m == v2, lane, E), axis=-1, keepdims=True)

    tp_ref[...] = jnp.concatenate([v1, v2], axis=1)
    ti_ref[...] = jnp.concatenate([i1, i2], axis=1)
    onehot1 = (lane == i1).astype(jnp.float32)
    onehot2 = (lane == i2).astype(jnp.float32)
    comb_ref[...] = v1 * onehot1 + v2 * onehot2

    mask = onehot1 + onehot2
    fraction = jnp.mean(mask, axis=0, keepdims=True)
    mean_prob = jnp.mean(p, axis=0, keepdims=True)
    lbl = E * jnp.sum(fraction * mean_prob, axis=1, keepdims=True)
    zm = jnp.maximum(v1, v2)
    lse = zm + jnp.log(jnp.exp(v1 - zm) + jnp.exp(v2 - zm))
    zl = jnp.mean(lse * lse, axis=0, keepdims=True)
    aux_ref[...] = lbl * ROUTER_AUX_COEF + zl * ROUTER_Z_COEF


def _ffn_body(x_ref, w1_ref, b1_ref, w2_ref, b2_ref, comb_ref, o_ref):
    e = pl.program_id(0)

    @pl.when(e == 0)
    def _():
        o_ref[...] = jnp.zeros_like(o_ref)

    lane = jax.lax.broadcasted_iota(jnp.int32, (T, E), 1)
    factor = jnp.sum(
        jnp.where(lane == e, comb_ref[...], 0.0), axis=1, keepdims=True)

    xb = x_ref[...].astype(jnp.bfloat16)
    w1b = w1_ref[0].astype(jnp.bfloat16)
    h = jax.lax.dot_general(
        xb, w1b, (((1,), (0,)), ((), ())),
        preferred_element_type=jnp.float32) + b1_ref[0]
    h = jax.nn.gelu(h)
    w2b = w2_ref[0].astype(jnp.bfloat16)
    y = jax.lax.dot_general(
        h.astype(jnp.bfloat16), w2b, (((1,), (0,)), ((), ())),
        preferred_element_type=jnp.float32) + b2_ref[0]
    o_ref[...] += factor * y


@jax.jit
def kernel(hidden_states, router_w, router_b, w1, b1, w2, b2):
    tokens = hidden_states.reshape(T, H)

    top_probs, top_idx, combine, aux = pl.pallas_call(
        _router_body,
        out_shape=(
            jax.ShapeDtypeStruct((T, K), jnp.float32),
            jax.ShapeDtypeStruct((T, K), jnp.int32),
            jax.ShapeDtypeStruct((T, E), jnp.float32),
            jax.ShapeDtypeStruct((1, 1), jnp.float32),
        ),
    )(tokens, router_w, router_b)

    out = pl.pallas_call(
        _ffn_body,
        grid=(E,),
        in_specs=[
            pl.BlockSpec((T, H), lambda e: (0, 0)),
            pl.BlockSpec((1, H, DFF), lambda e: (e, 0, 0)),
            pl.BlockSpec((1, 1, DFF), lambda e: (e, 0, 0)),
            pl.BlockSpec((1, DFF, H), lambda e: (e, 0, 0)),
            pl.BlockSpec((1, 1, H), lambda e: (e, 0, 0)),
            pl.BlockSpec((T, E), lambda e: (0, 0)),
        ],
        out_specs=pl.BlockSpec((T, H), lambda e: (0, 0)),
        out_shape=jax.ShapeDtypeStruct((T, H), jnp.float32),
    )(tokens, w1, b1.reshape(E, 1, DFF), w2, b2.reshape(E, 1, H), combine)

    output = out.reshape(B, S, H)
    aux_loss = aux[0, 0]
    route_probs = top_probs.reshape(B, S, K)
    route_indices = top_idx.reshape(B, S, K)
    return (output, aux_loss, route_probs, route_indices)
